# baseline (speedup 1.0000x reference)
import jax
import jax.numpy as jnp
import numpy as np
from jax import lax
from jax.experimental import pallas as pl
from jax.experimental.pallas import tpu as pltpu
from jax.experimental.pallas import tpu_sc as plsc

_B = 4096
_F = 26
_ED = 8
_SPARSE = _F * _ED
_N = _B * _F
_NW = 32
_PW = _N // _NW
_PWE = _PW * _ED
_MAIN = 999936
_FLAT = _MAIN * _ED


def _sc_gather_kernel(idx_hbm, tidx_hbm, wmain, dmain, wtail, dtail,
                      wide_out, deep_out,
                      idx_v, tidx_v, wtmp, dtmp, wsum, wtail_v, dtail_v,
                      sem_w, sem_d):
    info = plsc.get_sparse_core_info()
    nc = info.num_cores
    wid = lax.axis_index("s") * nc + lax.axis_index("c")
    base = wid * _PWE

    pltpu.sync_copy(idx_hbm.at[pl.ds(base, _PWE)], idx_v)
    pltpu.sync_copy(tidx_hbm.at[pl.ds(base, _PWE)], tidx_v)
    pltpu.sync_copy(wtail, wtail_v)
    pltpu.sync_copy(dtail, dtail_v)

    copies = []
    for c in range(_ED):
        sl = pl.ds(c * _PW, _PW)
        copies.append(pltpu.async_copy(wmain.at[idx_v.at[sl]],
                                       wtmp.at[sl], sem_w))
        copies.append(pltpu.async_copy(dmain.at[idx_v.at[sl]],
                                       dtmp.at[sl], sem_d))
    for cp in copies:
        cp.wait()

    def tail_body(g, carry):
        sl = pl.ds(g * 16, 16)
        tv = tidx_v[sl]
        m = tv > 0
        ti = tv - 1
        wv = plsc.load_gather(wtail_v, [ti], mask=m)
        dv = plsc.load_gather(dtail_v, [ti], mask=m)
        wtmp[sl] = jnp.where(m, wv, wtmp[sl])
        dtmp[sl] = jnp.where(m, dv, dtmp[sl])
        return carry

    lax.fori_loop(0, _PWE // 16, tail_body, 0)

    lanes8 = lax.iota(jnp.int32, 16) * _ED

    def wsum_body(g, carry):
        base_i = lanes8 + g * 128
        acc = plsc.load_gather(wtmp, [base_i])
        for c in range(1, _ED):
            acc = acc + plsc.load_gather(wtmp, [base_i + c])
        wsum[pl.ds(g * 16, 16)] = acc
        return carry

    lax.fori_loop(0, _PW // 16, wsum_body, 0)

    pltpu.sync_copy(wsum, wide_out.at[pl.ds(wid * _PW, _PW)])
    pltpu.sync_copy(dtmp, deep_out.at[pl.ds(base, _PWE)])


def _sc_gather(idx1d, tidx1d, wmain, dmain, wtail, dtail):
    mesh = plsc.VectorSubcoreMesh(core_axis_name="c", subcore_axis_name="s")
    f = pl.kernel(
        _sc_gather_kernel,
        out_type=[
            jax.ShapeDtypeStruct((_N,), jnp.float32),
            jax.ShapeDtypeStruct((_N * _ED,), jnp.float32),
        ],
        mesh=mesh,
        scratch_types=[
            pltpu.VMEM((_PWE,), jnp.int32),
            pltpu.VMEM((_PWE,), jnp.int32),
            pltpu.VMEM((_PWE,), jnp.float32),
            pltpu.VMEM((_PWE,), jnp.float32),
            pltpu.VMEM((_PW,), jnp.float32),
            pltpu.VMEM((8 * 64,), jnp.float32),
            pltpu.VMEM((8 * 64,), jnp.float32),
            pltpu.SemaphoreType.DMA,
            pltpu.SemaphoreType.DMA,
        ],
        compiler_params=pltpu.CompilerParams(use_tc_tiling_on_sc=False,
                                             needs_layout_passes=False),
    )
    return f(idx1d, tidx1d, wmain, dmain, wtail, dtail)


def _tc_mlp_kernel(deep_ref, wide_ref, g_ref, bta_ref,
                   w0_ref, b0_ref, w1_ref, b1_ref, w2_ref, b2_ref,
                   w3_ref, b3_ref, out_ref):
    wide_out = jnp.sum(wide_ref[...], axis=1, keepdims=True)
    h = deep_ref[...]
    mu = jnp.mean(h, axis=1, keepdims=True)
    var = jnp.mean(jnp.square(h - mu), axis=1, keepdims=True)
    h = (h - mu) * lax.rsqrt(var + 1e-5) * g_ref[...] + bta_ref[...]
    h = jnp.maximum(jnp.dot(h, w0_ref[...], preferred_element_type=jnp.float32)
                    + b0_ref[...], 0.0)
    h = jnp.maximum(jnp.dot(h, w1_ref[...], preferred_element_type=jnp.float32)
                    + b1_ref[...], 0.0)
    h = jnp.maximum(jnp.dot(h, w2_ref[...], preferred_element_type=jnp.float32)
                    + b2_ref[...], 0.0)
    dnn = jnp.sum(h * w3_ref[...], axis=1, keepdims=True) + b3_ref[...]
    out_ref[...] = jax.nn.sigmoid(wide_out + dnn)


def _tc_mlp(deep_emb, wide_f, ln_gamma, ln_beta,
            W0, b0, W1, b1, W2, b2, W3, b3):
    bb = 1024
    grid = (_B // bb,)
    full = lambda shape: pl.BlockSpec(shape, lambda i: (0, 0))
    return pl.pallas_call(
        _tc_mlp_kernel,
        grid=grid,
        in_specs=[
            pl.BlockSpec((bb, _SPARSE), lambda i: (i, 0)),
            pl.BlockSpec((bb, _F), lambda i: (i, 0)),
            full((1, _SPARSE)),
            full((1, _SPARSE)),
            full((_SPARSE, 512)),
            full((1, 512)),
            full((512, 256)),
            full((1, 256)),
            full((256, 128)),
            full((1, 128)),
            full((1, 128)),
            full((1, 1)),
        ],
        out_specs=pl.BlockSpec((bb, 1), lambda i: (i, 0)),
        out_shape=jax.ShapeDtypeStruct((_B, 1), jnp.float32),
    )(deep_emb, wide_f, ln_gamma.reshape(1, -1), ln_beta.reshape(1, -1),
      W0, b0.reshape(1, -1), W1, b1.reshape(1, -1), W2, b2.reshape(1, -1),
      W3.reshape(1, -1), b3.reshape(1, 1))


def _flat_view(table):
    return (table[:_MAIN].T.reshape(_ED, _MAIN // 128, 128)
            .transpose(1, 0, 2).reshape(-1))


@jax.jit
def kernel(x, wide_table, deep_table, ln_gamma, ln_beta,
           W0, b0, W1, b1, W2, b2, W3, b3):
    xw = x.reshape(_NW, _PW, 1)
    tail = xw >= _MAIN
    qbase = (xw >> 7) * 1024 + (xw & 127)
    cr = jnp.arange(_ED, dtype=jnp.int32).reshape(1, 1, _ED)
    idx1d = jnp.where(tail, 0, qbase + cr * 128).reshape(-1)
    tidx1d = jnp.where(tail, (xw - _MAIN) + 64 * cr + 1, 0).reshape(-1)
    wtail = wide_table[_MAIN:].T.reshape(-1)
    dtail = deep_table[_MAIN:].T.reshape(-1)

    wide_c, deep_flat = _sc_gather(
        idx1d, tidx1d, _flat_view(wide_table), _flat_view(deep_table),
        wtail, dtail)

    deep_emb = deep_flat.reshape(_B, _SPARSE)
    wide_f = wide_c.reshape(_B, _F)
    return _tc_mlp(deep_emb, wide_f, ln_gamma, ln_beta,
                   W0, b0, W1, b1, W2, b2, W3, b3)


# speedup vs baseline: 1.5539x; 1.5539x over previous
import jax
import jax.numpy as jnp
import numpy as np
from jax import lax
from jax.experimental import pallas as pl
from jax.experimental.pallas import tpu as pltpu
from jax.experimental.pallas import tpu_sc as plsc

_B = 4096
_F = 26
_ED = 8
_SPARSE = _F * _ED
_N = _B * _F
_NW = 32
_PW = _N // _NW
_PWE = _PW * _ED
_MAIN = 999936
_FLAT = _MAIN * _ED


def _sc_gather_kernel(idx_hbm, tidx_hbm, wmain, dmain, wtail, dtail,
                      wide_out, deep_out,
                      idx_v, tidx_v, wtmp, dtmp, wsum, wtail_v, dtail_v,
                      sem_w, sem_d):
    info = plsc.get_sparse_core_info()
    nc = info.num_cores
    wid = lax.axis_index("s") * nc + lax.axis_index("c")
    base = wid * _PWE

    pltpu.sync_copy(idx_hbm.at[pl.ds(base, _PWE)], idx_v)
    pltpu.sync_copy(tidx_hbm.at[pl.ds(base, _PWE)], tidx_v)
    pltpu.sync_copy(wtail, wtail_v)
    pltpu.sync_copy(dtail, dtail_v)

    copies = []
    for c in range(_ED):
        sl = pl.ds(c * _PW, _PW)
        copies.append(pltpu.async_copy(wmain.at[idx_v.at[sl]],
                                       wtmp.at[sl], sem_w))
        copies.append(pltpu.async_copy(dmain.at[idx_v.at[sl]],
                                       dtmp.at[sl], sem_d))
    for cp in copies:
        cp.wait()

    def tail_body(g, carry):
        sl = pl.ds(g * 16, 16)
        tv = tidx_v[sl]
        m = tv > 0
        ti = tv - 1
        wv = plsc.load_gather(wtail_v, [ti], mask=m)
        dv = plsc.load_gather(dtail_v, [ti], mask=m)
        wtmp[sl] = jnp.where(m, wv, wtmp[sl])
        dtmp[sl] = jnp.where(m, dv, dtmp[sl])
        return carry

    lax.fori_loop(0, _PWE // 16, tail_body, 0)

    def wsum_body(g, carry):
        sl = pl.ds(g * 16, 16)
        acc = wtmp[sl]
        for c in range(1, _ED):
            acc = acc + wtmp[pl.ds(c * _PW + g * 16, 16)]
        wsum[sl] = acc
        return carry

    lax.fori_loop(0, _PW // 16, wsum_body, 0)
    pltpu.sync_copy(wsum, wide_out.at[pl.ds(wid * _PW, _PW)])

    lanes = lax.iota(jnp.int32, 16)

    def reord_body(g, carry):
        pv = lanes + g * 16
        src = (pv & 7) * _PW + (pv >> 3)
        wtmp[pl.ds(g * 16, 16)] = plsc.load_gather(dtmp, [src])
        return carry

    lax.fori_loop(0, _PWE // 16, reord_body, 0)
    pltpu.sync_copy(wtmp, deep_out.at[pl.ds(base, _PWE)])


def _sc_gather(idx1d, tidx1d, wmain, dmain, wtail, dtail):
    mesh = plsc.VectorSubcoreMesh(core_axis_name="c", subcore_axis_name="s")
    f = pl.kernel(
        _sc_gather_kernel,
        out_type=[
            jax.ShapeDtypeStruct((_N,), jnp.float32),
            jax.ShapeDtypeStruct((_N * _ED,), jnp.float32),
        ],
        mesh=mesh,
        scratch_types=[
            pltpu.VMEM((_PWE,), jnp.int32),
            pltpu.VMEM((_PWE,), jnp.int32),
            pltpu.VMEM((_PWE,), jnp.float32),
            pltpu.VMEM((_PWE,), jnp.float32),
            pltpu.VMEM((_PW,), jnp.float32),
            pltpu.VMEM((8 * 64,), jnp.float32),
            pltpu.VMEM((8 * 64,), jnp.float32),
            pltpu.SemaphoreType.DMA,
            pltpu.SemaphoreType.DMA,
        ],
        compiler_params=pltpu.CompilerParams(use_tc_tiling_on_sc=False,
                                             needs_layout_passes=False),
    )
    return f(idx1d, tidx1d, wmain, dmain, wtail, dtail)


def _tc_mlp_kernel(deep_ref, wide_ref, g_ref, bta_ref,
                   w0_ref, b0_ref, w1_ref, b1_ref, w2_ref, b2_ref,
                   w3_ref, b3_ref, out_ref):
    wide_out = jnp.sum(wide_ref[...], axis=1, keepdims=True)
    h = deep_ref[...]
    mu = jnp.mean(h, axis=1, keepdims=True)
    var = jnp.mean(jnp.square(h - mu), axis=1, keepdims=True)
    h = (h - mu) * lax.rsqrt(var + 1e-5) * g_ref[...] + bta_ref[...]
    h = jnp.maximum(jnp.dot(h, w0_ref[...], preferred_element_type=jnp.float32)
                    + b0_ref[...], 0.0)
    h = jnp.maximum(jnp.dot(h, w1_ref[...], preferred_element_type=jnp.float32)
                    + b1_ref[...], 0.0)
    h = jnp.maximum(jnp.dot(h, w2_ref[...], preferred_element_type=jnp.float32)
                    + b2_ref[...], 0.0)
    dnn = jnp.sum(h * w3_ref[...], axis=1, keepdims=True) + b3_ref[...]
    out_ref[...] = jax.nn.sigmoid(wide_out + dnn)


def _tc_mlp(deep_emb, wide_f, ln_gamma, ln_beta,
            W0, b0, W1, b1, W2, b2, W3, b3):
    bb = 1024
    grid = (_B // bb,)
    full = lambda shape: pl.BlockSpec(shape, lambda i: (0, 0))
    return pl.pallas_call(
        _tc_mlp_kernel,
        grid=grid,
        in_specs=[
            pl.BlockSpec((bb, _SPARSE), lambda i: (i, 0)),
            pl.BlockSpec((bb, _F), lambda i: (i, 0)),
            full((1, _SPARSE)),
            full((1, _SPARSE)),
            full((_SPARSE, 512)),
            full((1, 512)),
            full((512, 256)),
            full((1, 256)),
            full((256, 128)),
            full((1, 128)),
            full((1, 128)),
            full((1, 1)),
        ],
        out_specs=pl.BlockSpec((bb, 1), lambda i: (i, 0)),
        out_shape=jax.ShapeDtypeStruct((_B, 1), jnp.float32),
    )(deep_emb, wide_f, ln_gamma.reshape(1, -1), ln_beta.reshape(1, -1),
      W0, b0.reshape(1, -1), W1, b1.reshape(1, -1), W2, b2.reshape(1, -1),
      W3.reshape(1, -1), b3.reshape(1, 1))


def _flat_view(table):
    return (table[:_MAIN].T.reshape(_ED, _MAIN // 128, 128)
            .transpose(1, 0, 2).reshape(-1))


@jax.jit
def kernel(x, wide_table, deep_table, ln_gamma, ln_beta,
           W0, b0, W1, b1, W2, b2, W3, b3):
    xw = x.reshape(_NW, 1, _PW)
    tail = xw >= _MAIN
    qbase = (xw >> 7) * 1024 + (xw & 127)
    cr = jnp.arange(_ED, dtype=jnp.int32).reshape(1, _ED, 1)
    idx1d = jnp.where(tail, 0, qbase + cr * 128).reshape(-1)
    tidx1d = jnp.where(tail, (xw - _MAIN) + 64 * cr + 1, 0).reshape(-1)
    wtail = wide_table[_MAIN:].T.reshape(-1)
    dtail = deep_table[_MAIN:].T.reshape(-1)

    wide_c, deep_flat = _sc_gather(
        idx1d, tidx1d, _flat_view(wide_table), _flat_view(deep_table),
        wtail, dtail)

    deep_emb = deep_flat.reshape(_B, _SPARSE)
    wide_f = wide_c.reshape(_B, _F)
    return _tc_mlp(deep_emb, wide_f, ln_gamma, ln_beta,
                   W0, b0, W1, b1, W2, b2, W3, b3)
